# baseline (speedup 1.0000x reference)
import functools

import jax
import jax.numpy as jnp
from jax import lax
from jax.experimental import pallas as pl
from jax.experimental.pallas import tpu as pltpu
from jax.experimental.pallas import tpu_sc as plsc

T = 32768
E = 64
CAP = 512.0
BLK = 256
NBLK = T // BLK
NW = 32
TPW = T // NW


@functools.lru_cache(maxsize=1)
def _gumbel():
    u = jax.random.uniform(jax.random.key(12345), (T, E), minval=1e-6, maxval=1.0 - 1e-6)
    return -jnp.log(-jnp.log(u))


def _pass1_body(x_ref, wg_ref, gum_ref, i1_ref, i2_ref, g1_ref, g2_ref,
                r1_ref, r2_ref, cnt_ref, stats_ref, tril_ref):
    pid = pl.program_id(0)

    @pl.when(pid == 0)
    def _init():
        cnt_ref[...] = jnp.zeros((1, 2 * E), jnp.float32)
        stats_ref[...] = jnp.zeros((1, 2 * E), jnp.float32)
        tr = lax.broadcasted_iota(jnp.int32, (BLK, BLK), 0)
        tc = lax.broadcasted_iota(jnp.int32, (BLK, BLK), 1)
        tril_ref[...] = (tr > tc).astype(jnp.float32)

    logits = lax.dot_general(x_ref[...], wg_ref[...],
                             (((1,), (1,)), ((), ())),
                             preferred_element_type=jnp.float32)
    m = jnp.max(logits, axis=1, keepdims=True)
    ex = jnp.exp(logits - m)
    zinv = 1.0 / jnp.sum(ex, axis=1, keepdims=True)

    iota_e = lax.broadcasted_iota(jnp.int32, (BLK, E), 1)
    i1 = jnp.min(jnp.where(logits == m, iota_e, 127), axis=1)
    oh1 = iota_e == i1[:, None]

    noisy = jnp.where(oh1, -jnp.inf, logits + gum_ref[...])
    nm = jnp.max(noisy, axis=1, keepdims=True)
    i2 = jnp.min(jnp.where(noisy == nm, iota_e, 127), axis=1)
    oh2 = iota_e == i2[:, None]

    ohcat = jnp.concatenate(
        [oh1.astype(jnp.float32), oh2.astype(jnp.float32)], axis=1)
    pre = lax.dot_general(tril_ref[...], ohcat, (((1,), (0,)), ((), ())),
                          preferred_element_type=jnp.float32)
    cnt = cnt_ref[...]
    ranked = ohcat * (cnt + pre)
    r1_ref[...] = jnp.sum(ranked[:, :E], axis=1)
    r2_ref[...] = jnp.sum(ranked[:, E:], axis=1)

    zex = jnp.where(oh1, ex, 0.0)
    zex2 = jnp.where(oh2, ex, 0.0)
    g1_ref[...] = jnp.sum(zex, axis=1) * zinv[:, 0]
    g2_ref[...] = jnp.sum(zex2, axis=1) * zinv[:, 0]
    i1_ref[...] = i1.astype(jnp.float32)
    i2_ref[...] = i2.astype(jnp.float32)

    cnt_ref[...] = cnt + jnp.sum(ohcat, axis=0)[None, :]
    gsum = jnp.sum(ex * zinv, axis=0)
    stats_ref[0, :E] = stats_ref[0, :E] + gsum


_pass1 = pl.pallas_call(
    _pass1_body,
    grid=(NBLK,),
    in_specs=[
        pl.BlockSpec((BLK, 1024), lambda i: (i, 0)),
        pl.BlockSpec((E, 1024), lambda i: (0, 0)),
        pl.BlockSpec((BLK, E), lambda i: (i, 0)),
    ],
    out_specs=[pl.BlockSpec((BLK,), lambda i: (i,))] * 6
    + [pl.BlockSpec((1, 2 * E), lambda i: (0, 0))] * 2,
    out_shape=[jax.ShapeDtypeStruct((T,), jnp.float32)] * 6
    + [jax.ShapeDtypeStruct((1, 2 * E), jnp.float32)] * 2,
    scratch_shapes=[pltpu.VMEM((BLK, BLK), jnp.float32)],
)


def _pass2_body(i1_hbm, i2_hbm, g1_hbm, g2_hbm, r1_hbm, r2_hbm, c1tot_hbm,
                out_hbm, i1_v, i2_v, g1_v, g2_v, r1_v, r2_v, c1_v, out_v):
    wid = lax.axis_index("s") * 2 + lax.axis_index("c")
    base = wid * TPW
    pltpu.sync_copy(i1_hbm.at[pl.ds(base, TPW)], i1_v)
    pltpu.sync_copy(i2_hbm.at[pl.ds(base, TPW)], i2_v)
    pltpu.sync_copy(g1_hbm.at[pl.ds(base, TPW)], g1_v)
    pltpu.sync_copy(g2_hbm.at[pl.ds(base, TPW)], g2_v)
    pltpu.sync_copy(r1_hbm.at[pl.ds(base, TPW)], r1_v)
    pltpu.sync_copy(r2_hbm.at[pl.ds(base, TPW)], r2_v)
    pltpu.sync_copy(c1tot_hbm, c1_v)

    zeros16 = jnp.zeros((16,), jnp.float32)

    def _zero(k, _):
        out_v[pl.ds(k * 16, 16)] = zeros16
        return _

    lax.fori_loop(0, TPW * E // 16, _zero, None)

    lane = lax.broadcasted_iota(jnp.int32, (16,), 0)

    def _grp(g, _):
        sl = pl.ds(g * 16, 16)
        i1i = i1_v[sl].astype(jnp.int32)
        i2i = i2_v[sl].astype(jnp.int32)
        keep1 = r1_v[sl] < CAP
        c1at2 = plsc.load_gather(c1_v, [i2i])
        keep2 = (c1at2 + r2_v[sl]) < CAP
        row = (g * 16 + lane) * E
        plsc.store_scatter(out_v, [row + i1i], g1_v[sl], mask=keep1)
        plsc.store_scatter(out_v, [row + i2i], g2_v[sl], mask=keep2)
        return _

    lax.fori_loop(0, TPW // 16, _grp, None)

    pltpu.sync_copy(out_v, out_hbm.at[pl.ds(base * E, TPW * E)])


@functools.lru_cache(maxsize=1)
def _pass2():
    return pl.kernel(
        _pass2_body,
        out_type=jax.ShapeDtypeStruct((T * E,), jnp.float32),
        mesh=plsc.VectorSubcoreMesh(core_axis_name="c", subcore_axis_name="s"),
        scratch_types=[pltpu.VMEM((TPW,), jnp.float32)] * 6
        + [pltpu.VMEM((E,), jnp.float32), pltpu.VMEM((TPW * E,), jnp.float32)],
        compiler_params=pltpu.CompilerParams(needs_layout_passes=False),
    )


def kernel(x, wg_weight):
    i1, i2, g1, g2, r1, r2, cnt, stats = _pass1(x, wg_weight, _gumbel())
    c1tot = cnt[0, :E]
    flat = _pass2()(i1, i2, g1, g2, r1, r2, c1tot)
    combine = flat.reshape(T, E)
    l_aux = jnp.sum((stats[0, :E] / T) * (c1tot / T)) * E
    return (l_aux, combine)


# speedup vs baseline: 2.3652x; 2.3652x over previous
import functools

import jax
import jax.numpy as jnp
from jax import lax
from jax.experimental import pallas as pl
from jax.experimental.pallas import tpu as pltpu
from jax.experimental.pallas import tpu_sc as plsc

T = 32768
E = 64
CAP = 512.0
BLK = 256
NBLK = T // BLK
NW = 32
TPW = T // NW


@functools.lru_cache(maxsize=1)
def _gumbel():
    u = jax.random.uniform(jax.random.key(12345), (T, E), minval=1e-6, maxval=1.0 - 1e-6)
    return jnp.transpose(-jnp.log(-jnp.log(u)))


def _pass1_body(x_ref, wg_ref, gum_ref, i1_ref, i2_ref, g1_ref, g2_ref,
                r1_ref, r2_ref, cnt_ref, me_ref, s_ref, stats_ref):
    pid = pl.program_id(0)

    @pl.when(pid == 0)
    def _init():
        cnt_ref[...] = jnp.zeros((2 * E, 1), jnp.float32)
        stats_ref[...] = jnp.zeros((E, BLK), jnp.float32)
        tr = lax.broadcasted_iota(jnp.int32, (BLK, BLK), 0)
        tc = lax.broadcasted_iota(jnp.int32, (BLK, BLK), 1)
        s_ref[...] = (tr < tc).astype(jnp.float32)

    logits = lax.dot_general(wg_ref[...], x_ref[...],
                             (((1,), (1,)), ((), ())),
                             preferred_element_type=jnp.float32)
    m = jnp.max(logits, axis=0, keepdims=True)
    ex = jnp.exp(logits - m)
    zinv = 1.0 / jnp.sum(ex, axis=0, keepdims=True)

    iota_e = lax.broadcasted_iota(jnp.int32, (E, BLK), 0)
    i1 = jnp.min(jnp.where(logits == m, iota_e, 127), axis=0, keepdims=True)
    oh1 = iota_e == i1

    noisy = jnp.where(oh1, -jnp.inf, logits + gum_ref[...])
    nm = jnp.max(noisy, axis=0, keepdims=True)
    i2 = jnp.min(jnp.where(noisy == nm, iota_e, 127), axis=0, keepdims=True)
    oh2 = iota_e == i2

    ohcat = jnp.concatenate(
        [oh1.astype(jnp.float32), oh2.astype(jnp.float32)], axis=0)
    pre = lax.dot_general(ohcat, s_ref[...], (((1,), (0,)), ((), ())),
                          preferred_element_type=jnp.float32)
    cnt = cnt_ref[...]
    ranked = ohcat * (cnt + pre)
    r1_ref[...] = jnp.sum(ranked[:E], axis=0, keepdims=True)
    r2_ref[...] = jnp.sum(ranked[E:], axis=0, keepdims=True)

    probs = ex * zinv
    g1_ref[...] = jnp.sum(jnp.where(oh1, probs, 0.0), axis=0, keepdims=True)
    g2_ref[...] = jnp.sum(jnp.where(oh2, probs, 0.0), axis=0, keepdims=True)
    i1_ref[...] = i1.astype(jnp.float32)
    i2_ref[...] = i2.astype(jnp.float32)

    cnt_ref[...] = cnt + pre[:, BLK - 1:BLK] + ohcat[:, BLK - 1:BLK]
    stats_ref[...] = stats_ref[...] + probs

    @pl.when(pid == NBLK - 1)
    def _fin():
        me_ref[...] = jnp.sum(stats_ref[...], axis=1, keepdims=True)


_pass1 = pl.pallas_call(
    _pass1_body,
    grid=(NBLK,),
    in_specs=[
        pl.BlockSpec((BLK, 1024), lambda i: (i, 0)),
        pl.BlockSpec((E, 1024), lambda i: (0, 0)),
        pl.BlockSpec((E, BLK), lambda i: (0, i)),
    ],
    out_specs=[pl.BlockSpec((1, BLK), lambda i: (0, i))] * 6
    + [pl.BlockSpec((2 * E, 1), lambda i: (0, 0)),
       pl.BlockSpec((E, 1), lambda i: (0, 0))],
    out_shape=[jax.ShapeDtypeStruct((1, T), jnp.float32)] * 6
    + [jax.ShapeDtypeStruct((2 * E, 1), jnp.float32),
       jax.ShapeDtypeStruct((E, 1), jnp.float32)],
    scratch_shapes=[pltpu.VMEM((BLK, BLK), jnp.float32),
                    pltpu.VMEM((E, BLK), jnp.float32)],
)


def _pass2_body(i1_hbm, i2_hbm, g1_hbm, g2_hbm, r1_hbm, r2_hbm, c1tot_hbm,
                out_hbm, i1_v, i2_v, g1_v, g2_v, r1_v, r2_v, c1_v, out_v):
    wid = lax.axis_index("s") * 2 + lax.axis_index("c")
    base = wid * TPW
    pltpu.sync_copy(i1_hbm.at[pl.ds(base, TPW)], i1_v)
    pltpu.sync_copy(i2_hbm.at[pl.ds(base, TPW)], i2_v)
    pltpu.sync_copy(g1_hbm.at[pl.ds(base, TPW)], g1_v)
    pltpu.sync_copy(g2_hbm.at[pl.ds(base, TPW)], g2_v)
    pltpu.sync_copy(r1_hbm.at[pl.ds(base, TPW)], r1_v)
    pltpu.sync_copy(r2_hbm.at[pl.ds(base, TPW)], r2_v)
    pltpu.sync_copy(c1tot_hbm, c1_v)

    zeros16 = jnp.zeros((16,), jnp.float32)

    def _zero(k, _):
        out_v[pl.ds(k * 16, 16)] = zeros16
        return _

    lax.fori_loop(0, TPW * E // 16, _zero, None)

    lane = lax.broadcasted_iota(jnp.int32, (16,), 0)

    def _grp(g, _):
        sl = pl.ds(g * 16, 16)
        i1i = i1_v[sl].astype(jnp.int32)
        i2i = i2_v[sl].astype(jnp.int32)
        keep1 = r1_v[sl] < CAP
        c1at2 = plsc.load_gather(c1_v, [i2i])
        keep2 = (c1at2 + r2_v[sl]) < CAP
        row = (g * 16 + lane) * E
        plsc.store_scatter(out_v, [row + i1i], g1_v[sl], mask=keep1)
        plsc.store_scatter(out_v, [row + i2i], g2_v[sl], mask=keep2)
        return _

    lax.fori_loop(0, TPW // 16, _grp, None)

    pltpu.sync_copy(out_v, out_hbm.at[pl.ds(base * E, TPW * E)])


@functools.lru_cache(maxsize=1)
def _pass2():
    return pl.kernel(
        _pass2_body,
        out_type=jax.ShapeDtypeStruct((T * E,), jnp.float32),
        mesh=plsc.VectorSubcoreMesh(core_axis_name="c", subcore_axis_name="s"),
        scratch_types=[pltpu.VMEM((TPW,), jnp.float32)] * 6
        + [pltpu.VMEM((E,), jnp.float32), pltpu.VMEM((TPW * E,), jnp.float32)],
        compiler_params=pltpu.CompilerParams(needs_layout_passes=False),
    )


def kernel(x, wg_weight):
    i1, i2, g1, g2, r1, r2, cnt, me = _pass1(x, wg_weight, _gumbel())
    c1tot = cnt[:E, 0]
    flat = _pass2()(i1.reshape(T), i2.reshape(T), g1.reshape(T),
                    g2.reshape(T), r1.reshape(T), r2.reshape(T), c1tot)
    combine = flat.reshape(T, E)
    l_aux = jnp.sum((me[:, 0] / T) * (c1tot / T)) * E
    return (l_aux, combine)


# speedup vs baseline: 2.8585x; 1.2086x over previous
import functools

import jax
import jax.numpy as jnp
from jax import lax
from jax.experimental import pallas as pl
from jax.experimental.pallas import tpu as pltpu
from jax.experimental.pallas import tpu_sc as plsc

T = 32768
E = 64
CAP = 512.0
BLK = 512
NBLK = T // BLK
NW = 32
TPW = T // NW


@functools.lru_cache(maxsize=1)
def _gumbel():
    u = jax.random.uniform(jax.random.key(12345), (T, E), minval=1e-6, maxval=1.0 - 1e-6)
    return jnp.transpose(-jnp.log(-jnp.log(u)))


def _pass1_body(x_ref, wg_ref, gum_ref, i1_ref, i2_ref, g1_ref, g2_ref,
                r1_ref, r2_ref, cnt_ref, me_ref, s_ref, stats_ref):
    pid = pl.program_id(0)

    @pl.when(pid == 0)
    def _init():
        cnt_ref[...] = jnp.zeros((2 * E, 1), jnp.float32)
        stats_ref[...] = jnp.zeros((E, BLK), jnp.float32)
        tr = lax.broadcasted_iota(jnp.int32, (BLK, BLK), 0)
        tc = lax.broadcasted_iota(jnp.int32, (BLK, BLK), 1)
        s_ref[...] = (tr < tc).astype(jnp.float32)

    logits = lax.dot_general(wg_ref[...], x_ref[...],
                             (((1,), (1,)), ((), ())),
                             preferred_element_type=jnp.float32)
    m = jnp.max(logits, axis=0, keepdims=True)
    ex = jnp.exp(logits - m)
    zinv = 1.0 / jnp.sum(ex, axis=0, keepdims=True)

    iota_e = lax.broadcasted_iota(jnp.int32, (E, BLK), 0)
    i1 = jnp.min(jnp.where(logits == m, iota_e, 127), axis=0, keepdims=True)
    oh1 = iota_e == i1

    noisy = jnp.where(oh1, -jnp.inf, logits + gum_ref[...])
    nm = jnp.max(noisy, axis=0, keepdims=True)
    i2 = jnp.min(jnp.where(noisy == nm, iota_e, 127), axis=0, keepdims=True)
    oh2 = iota_e == i2

    ohcat = jnp.concatenate(
        [oh1.astype(jnp.float32), oh2.astype(jnp.float32)], axis=0)
    pre = lax.dot_general(ohcat, s_ref[...], (((1,), (0,)), ((), ())),
                          preferred_element_type=jnp.float32)
    cnt = cnt_ref[...]
    ranked = ohcat * (cnt + pre)
    r1_ref[...] = jnp.sum(ranked[:E], axis=0, keepdims=True)
    r2_ref[...] = jnp.sum(ranked[E:], axis=0, keepdims=True)

    probs = ex * zinv
    g1_ref[...] = jnp.sum(jnp.where(oh1, probs, 0.0), axis=0, keepdims=True)
    g2_ref[...] = jnp.sum(jnp.where(oh2, probs, 0.0), axis=0, keepdims=True)
    i1_ref[...] = i1.astype(jnp.float32)
    i2_ref[...] = i2.astype(jnp.float32)

    cnt_ref[...] = cnt + pre[:, BLK - 1:BLK] + ohcat[:, BLK - 1:BLK]
    stats_ref[...] = stats_ref[...] + probs

    @pl.when(pid == NBLK - 1)
    def _fin():
        me_ref[...] = jnp.sum(stats_ref[...], axis=1, keepdims=True)


_pass1 = pl.pallas_call(
    _pass1_body,
    grid=(NBLK,),
    in_specs=[
        pl.BlockSpec((BLK, 1024), lambda i: (i, 0)),
        pl.BlockSpec((E, 1024), lambda i: (0, 0)),
        pl.BlockSpec((E, BLK), lambda i: (0, i)),
    ],
    out_specs=[pl.BlockSpec((1, BLK), lambda i: (0, i))] * 6
    + [pl.BlockSpec((2 * E, 1), lambda i: (0, 0)),
       pl.BlockSpec((E, 1), lambda i: (0, 0))],
    out_shape=[jax.ShapeDtypeStruct((1, T), jnp.float32)] * 6
    + [jax.ShapeDtypeStruct((2 * E, 1), jnp.float32),
       jax.ShapeDtypeStruct((E, 1), jnp.float32)],
    scratch_shapes=[pltpu.VMEM((BLK, BLK), jnp.float32),
                    pltpu.VMEM((E, BLK), jnp.float32)],
)


def _pass2_body(i1_hbm, i2_hbm, g1_hbm, g2_hbm, r1_hbm, r2_hbm, c1tot_hbm,
                out_hbm, i1_v, i2_v, g1_v, g2_v, r1_v, r2_v, c1_v, out_v):
    wid = lax.axis_index("s") * 2 + lax.axis_index("c")
    base = wid * TPW
    pltpu.sync_copy(i1_hbm.at[pl.ds(base, TPW)], i1_v)
    pltpu.sync_copy(i2_hbm.at[pl.ds(base, TPW)], i2_v)
    pltpu.sync_copy(g1_hbm.at[pl.ds(base, TPW)], g1_v)
    pltpu.sync_copy(g2_hbm.at[pl.ds(base, TPW)], g2_v)
    pltpu.sync_copy(r1_hbm.at[pl.ds(base, TPW)], r1_v)
    pltpu.sync_copy(r2_hbm.at[pl.ds(base, TPW)], r2_v)
    pltpu.sync_copy(c1tot_hbm, c1_v)

    zeros16 = jnp.zeros((16,), jnp.float32)

    def _zero(k, _):
        out_v[pl.ds(k * 16, 16)] = zeros16
        return _

    lax.fori_loop(0, TPW * E // 16, _zero, None)

    lane = lax.broadcasted_iota(jnp.int32, (16,), 0)

    def _grp(g, _):
        sl = pl.ds(g * 16, 16)
        i1i = i1_v[sl].astype(jnp.int32)
        i2i = i2_v[sl].astype(jnp.int32)
        keep1 = r1_v[sl] < CAP
        c1at2 = plsc.load_gather(c1_v, [i2i])
        keep2 = (c1at2 + r2_v[sl]) < CAP
        row = (g * 16 + lane) * E
        plsc.store_scatter(out_v, [row + i1i], g1_v[sl], mask=keep1)
        plsc.store_scatter(out_v, [row + i2i], g2_v[sl], mask=keep2)
        return _

    lax.fori_loop(0, TPW // 16, _grp, None)

    pltpu.sync_copy(out_v, out_hbm.at[pl.ds(base * E, TPW * E)])


@functools.lru_cache(maxsize=1)
def _pass2():
    return pl.kernel(
        _pass2_body,
        out_type=jax.ShapeDtypeStruct((T * E,), jnp.float32),
        mesh=plsc.VectorSubcoreMesh(core_axis_name="c", subcore_axis_name="s"),
        scratch_types=[pltpu.VMEM((TPW,), jnp.float32)] * 6
        + [pltpu.VMEM((E,), jnp.float32), pltpu.VMEM((TPW * E,), jnp.float32)],
        compiler_params=pltpu.CompilerParams(needs_layout_passes=False),
    )


def kernel(x, wg_weight):
    i1, i2, g1, g2, r1, r2, cnt, me = _pass1(x, wg_weight, _gumbel())
    c1tot = cnt[:E, 0]
    flat = _pass2()(i1.reshape(T), i2.reshape(T), g1.reshape(T),
                    g2.reshape(T), r1.reshape(T), r2.reshape(T), c1tot)
    combine = flat.reshape(T, E)
    l_aux = jnp.sum((me[:, 0] / T) * (c1tot / T)) * E
    return (l_aux, combine)


# speedup vs baseline: 2.8634x; 1.0017x over previous
import functools

import jax
import jax.numpy as jnp
from jax import lax
from jax.experimental import pallas as pl
from jax.experimental.pallas import tpu as pltpu
from jax.experimental.pallas import tpu_sc as plsc

T = 32768
E = 64
CAP = 512.0
BLK = 512
NBLK = T // BLK
NW = 32
TPW = T // NW


@functools.lru_cache(maxsize=1)
def _gumbel():
    u = jax.random.uniform(jax.random.key(12345), (T, E), minval=1e-6, maxval=1.0 - 1e-6)
    return jnp.transpose(-jnp.log(-jnp.log(u)))


def _pass1_body(x_ref, wg_ref, gum_ref, i1_ref, i2_ref, g1_ref, g2_ref,
                r1_ref, r2_ref, cnt_ref, me_ref, s_ref, stats_ref):
    pid = pl.program_id(0)

    @pl.when(pid == 0)
    def _init():
        cnt_ref[...] = jnp.zeros((2 * E, 1), jnp.float32)
        stats_ref[...] = jnp.zeros((E, BLK), jnp.float32)
        tr = lax.broadcasted_iota(jnp.int32, (BLK, BLK), 0)
        tc = lax.broadcasted_iota(jnp.int32, (BLK, BLK), 1)
        s_ref[...] = (tr < tc).astype(jnp.bfloat16)

    logits = lax.dot_general(wg_ref[...], x_ref[...],
                             (((1,), (1,)), ((), ())),
                             preferred_element_type=jnp.float32)
    m = jnp.max(logits, axis=0, keepdims=True)
    ex = jnp.exp(logits - m)
    zinv = 1.0 / jnp.sum(ex, axis=0, keepdims=True)

    iota_e = lax.broadcasted_iota(jnp.int32, (E, BLK), 0)
    i1 = jnp.min(jnp.where(logits == m, iota_e, 127), axis=0, keepdims=True)
    oh1 = iota_e == i1

    noisy = jnp.where(oh1, -jnp.inf, logits + gum_ref[...])
    nm = jnp.max(noisy, axis=0, keepdims=True)
    i2 = jnp.min(jnp.where(noisy == nm, iota_e, 127), axis=0, keepdims=True)
    oh2 = iota_e == i2

    ohb = jnp.concatenate(
        [oh1.astype(jnp.bfloat16), oh2.astype(jnp.bfloat16)], axis=0)
    pre = lax.dot_general(ohb, s_ref[...], (((1,), (0,)), ((), ())),
                          preferred_element_type=jnp.float32)
    ohcat = ohb.astype(jnp.float32)
    cnt = cnt_ref[...]
    ranked = ohcat * (cnt + pre)
    r1_ref[...] = jnp.sum(ranked[:E], axis=0, keepdims=True)
    r2_ref[...] = jnp.sum(ranked[E:], axis=0, keepdims=True)

    probs = ex * zinv
    g1_ref[...] = jnp.sum(jnp.where(oh1, probs, 0.0), axis=0, keepdims=True)
    g2_ref[...] = jnp.sum(jnp.where(oh2, probs, 0.0), axis=0, keepdims=True)
    i1_ref[...] = i1.astype(jnp.float32)
    i2_ref[...] = i2.astype(jnp.float32)

    cnt_ref[...] = cnt + pre[:, BLK - 1:BLK] + ohcat[:, BLK - 1:BLK]
    stats_ref[...] = stats_ref[...] + probs

    @pl.when(pid == NBLK - 1)
    def _fin():
        me_ref[...] = jnp.sum(stats_ref[...], axis=1, keepdims=True)


_pass1 = pl.pallas_call(
    _pass1_body,
    grid=(NBLK,),
    in_specs=[
        pl.BlockSpec((BLK, 1024), lambda i: (i, 0)),
        pl.BlockSpec((E, 1024), lambda i: (0, 0)),
        pl.BlockSpec((E, BLK), lambda i: (0, i)),
    ],
    out_specs=[pl.BlockSpec((1, BLK), lambda i: (0, i))] * 6
    + [pl.BlockSpec((2 * E, 1), lambda i: (0, 0)),
       pl.BlockSpec((E, 1), lambda i: (0, 0))],
    out_shape=[jax.ShapeDtypeStruct((1, T), jnp.float32)] * 6
    + [jax.ShapeDtypeStruct((2 * E, 1), jnp.float32),
       jax.ShapeDtypeStruct((E, 1), jnp.float32)],
    scratch_shapes=[pltpu.VMEM((BLK, BLK), jnp.bfloat16),
                    pltpu.VMEM((E, BLK), jnp.float32)],
)


def _pass2_body(i1_hbm, i2_hbm, g1_hbm, g2_hbm, r1_hbm, r2_hbm, c1tot_hbm,
                out_hbm, i1_v, i2_v, g1_v, g2_v, r1_v, r2_v, c1_v, out_v):
    wid = lax.axis_index("s") * 2 + lax.axis_index("c")
    base = wid * TPW
    pltpu.sync_copy(i1_hbm.at[pl.ds(base, TPW)], i1_v)
    pltpu.sync_copy(i2_hbm.at[pl.ds(base, TPW)], i2_v)
    pltpu.sync_copy(g1_hbm.at[pl.ds(base, TPW)], g1_v)
    pltpu.sync_copy(g2_hbm.at[pl.ds(base, TPW)], g2_v)
    pltpu.sync_copy(r1_hbm.at[pl.ds(base, TPW)], r1_v)
    pltpu.sync_copy(r2_hbm.at[pl.ds(base, TPW)], r2_v)
    pltpu.sync_copy(c1tot_hbm, c1_v)

    zeros16 = jnp.zeros((16,), jnp.float32)

    def _zero(k, _):
        out_v[pl.ds(k * 16, 16)] = zeros16
        return _

    lax.fori_loop(0, TPW * E // 16, _zero, None)

    lane = lax.broadcasted_iota(jnp.int32, (16,), 0)

    def _grp(g, _):
        sl = pl.ds(g * 16, 16)
        i1i = i1_v[sl].astype(jnp.int32)
        i2i = i2_v[sl].astype(jnp.int32)
        keep1 = r1_v[sl] < CAP
        c1at2 = plsc.load_gather(c1_v, [i2i])
        keep2 = (c1at2 + r2_v[sl]) < CAP
        row = (g * 16 + lane) * E
        plsc.store_scatter(out_v, [row + i1i], g1_v[sl], mask=keep1)
        plsc.store_scatter(out_v, [row + i2i], g2_v[sl], mask=keep2)
        return _

    lax.fori_loop(0, TPW // 16, _grp, None)

    pltpu.sync_copy(out_v, out_hbm.at[pl.ds(base * E, TPW * E)])


@functools.lru_cache(maxsize=1)
def _pass2():
    return pl.kernel(
        _pass2_body,
        out_type=jax.ShapeDtypeStruct((T * E,), jnp.float32),
        mesh=plsc.VectorSubcoreMesh(core_axis_name="c", subcore_axis_name="s"),
        scratch_types=[pltpu.VMEM((TPW,), jnp.float32)] * 6
        + [pltpu.VMEM((E,), jnp.float32), pltpu.VMEM((TPW * E,), jnp.float32)],
        compiler_params=pltpu.CompilerParams(needs_layout_passes=False),
    )


def kernel(x, wg_weight):
    i1, i2, g1, g2, r1, r2, cnt, me = _pass1(x, wg_weight, _gumbel())
    c1tot = cnt[:E, 0]
    flat = _pass2()(i1.reshape(T), i2.reshape(T), g1.reshape(T),
                    g2.reshape(T), r1.reshape(T), r2.reshape(T), c1tot)
    combine = flat.reshape(T, E)
    l_aux = jnp.sum((me[:, 0] / T) * (c1tot / T)) * E
    return (l_aux, combine)


# speedup vs baseline: 2.8734x; 1.0035x over previous
import functools

import jax
import jax.numpy as jnp
from jax import lax
from jax.experimental import pallas as pl
from jax.experimental.pallas import tpu as pltpu
from jax.experimental.pallas import tpu_sc as plsc

T = 32768
E = 64
CAP = 512.0
BLK = 512
NBLK = T // BLK
NW = 32
TPW = T // NW


@functools.lru_cache(maxsize=1)
def _gumbel():
    u = jax.random.uniform(jax.random.key(12345), (T, E), minval=1e-6, maxval=1.0 - 1e-6)
    return jnp.transpose(-jnp.log(-jnp.log(u)))


def _pass1_body(xa_ref, xb_ref, wg_ref, gum_ref, i1_ref, i2_ref, g1_ref,
                g2_ref, r1_ref, r2_ref, cnt_ref, me_ref, s_ref, stats_ref):
    pid = pl.program_id(0)

    @pl.when(pid == 0)
    def _init():
        cnt_ref[...] = jnp.zeros((2 * E, 1), jnp.float32)
        stats_ref[...] = jnp.zeros((E, BLK), jnp.float32)
        tr = lax.broadcasted_iota(jnp.int32, (BLK, BLK), 0)
        tc = lax.broadcasted_iota(jnp.int32, (BLK, BLK), 1)
        s_ref[...] = (tr < tc).astype(jnp.bfloat16)

    la = lax.dot_general(wg_ref[...], xa_ref[...],
                         (((1,), (1,)), ((), ())),
                         preferred_element_type=jnp.float32)
    lb = lax.dot_general(wg_ref[...], xb_ref[...],
                         (((1,), (1,)), ((), ())),
                         preferred_element_type=jnp.float32)
    logits = jnp.concatenate([la, lb], axis=1)
    m = jnp.max(logits, axis=0, keepdims=True)
    ex = jnp.exp(logits - m)
    zinv = 1.0 / jnp.sum(ex, axis=0, keepdims=True)

    iota_e = lax.broadcasted_iota(jnp.int32, (E, BLK), 0)
    i1 = jnp.min(jnp.where(logits == m, iota_e, 127), axis=0, keepdims=True)
    oh1 = iota_e == i1

    noisy = jnp.where(oh1, -jnp.inf, logits + gum_ref[...])
    nm = jnp.max(noisy, axis=0, keepdims=True)
    i2 = jnp.min(jnp.where(noisy == nm, iota_e, 127), axis=0, keepdims=True)
    oh2 = iota_e == i2

    ohb = jnp.concatenate(
        [oh1.astype(jnp.bfloat16), oh2.astype(jnp.bfloat16)], axis=0)
    pre = lax.dot_general(ohb, s_ref[...], (((1,), (0,)), ((), ())),
                          preferred_element_type=jnp.float32)
    ohcat = ohb.astype(jnp.float32)
    cnt = cnt_ref[...]
    ranked = ohcat * (cnt + pre)
    r1_ref[...] = jnp.sum(ranked[:E], axis=0, keepdims=True)
    r2_ref[...] = jnp.sum(ranked[E:], axis=0, keepdims=True)

    probs = ex * zinv
    g1_ref[...] = jnp.sum(jnp.where(oh1, probs, 0.0), axis=0, keepdims=True)
    g2_ref[...] = jnp.sum(jnp.where(oh2, probs, 0.0), axis=0, keepdims=True)
    i1_ref[...] = i1.astype(jnp.float32)
    i2_ref[...] = i2.astype(jnp.float32)

    cnt_ref[...] = cnt + pre[:, BLK - 1:BLK] + ohcat[:, BLK - 1:BLK]
    stats_ref[...] = stats_ref[...] + probs

    @pl.when(pid == NBLK - 1)
    def _fin():
        me_ref[...] = jnp.sum(stats_ref[...], axis=1, keepdims=True)


_pass1 = pl.pallas_call(
    _pass1_body,
    grid=(NBLK,),
    in_specs=[
        pl.BlockSpec((BLK // 2, 1024), lambda i: (2 * i, 0)),
        pl.BlockSpec((BLK // 2, 1024), lambda i: (2 * i + 1, 0)),
        pl.BlockSpec((E, 1024), lambda i: (0, 0)),
        pl.BlockSpec((E, BLK), lambda i: (0, i)),
    ],
    out_specs=[pl.BlockSpec((1, BLK), lambda i: (0, i))] * 6
    + [pl.BlockSpec((2 * E, 1), lambda i: (0, 0)),
       pl.BlockSpec((E, 1), lambda i: (0, 0))],
    out_shape=[jax.ShapeDtypeStruct((1, T), jnp.float32)] * 6
    + [jax.ShapeDtypeStruct((2 * E, 1), jnp.float32),
       jax.ShapeDtypeStruct((E, 1), jnp.float32)],
    scratch_shapes=[pltpu.VMEM((BLK, BLK), jnp.bfloat16),
                    pltpu.VMEM((E, BLK), jnp.float32)],
)


def _pass2_body(i1_hbm, i2_hbm, g1_hbm, g2_hbm, r1_hbm, r2_hbm, c1tot_hbm,
                out_hbm, i1_v, i2_v, g1_v, g2_v, r1_v, r2_v, c1_v, out_v):
    wid = lax.axis_index("s") * 2 + lax.axis_index("c")
    base = wid * TPW
    pltpu.sync_copy(i1_hbm.at[pl.ds(base, TPW)], i1_v)
    pltpu.sync_copy(i2_hbm.at[pl.ds(base, TPW)], i2_v)
    pltpu.sync_copy(g1_hbm.at[pl.ds(base, TPW)], g1_v)
    pltpu.sync_copy(g2_hbm.at[pl.ds(base, TPW)], g2_v)
    pltpu.sync_copy(r1_hbm.at[pl.ds(base, TPW)], r1_v)
    pltpu.sync_copy(r2_hbm.at[pl.ds(base, TPW)], r2_v)
    pltpu.sync_copy(c1tot_hbm, c1_v)

    zeros16 = jnp.zeros((16,), jnp.float32)

    def _zero(k, _):
        out_v[pl.ds(k * 16, 16)] = zeros16
        return _

    lax.fori_loop(0, TPW * E // 16, _zero, None)

    lane = lax.broadcasted_iota(jnp.int32, (16,), 0)

    def _grp(g, _):
        sl = pl.ds(g * 16, 16)
        i1i = i1_v[sl].astype(jnp.int32)
        i2i = i2_v[sl].astype(jnp.int32)
        keep1 = r1_v[sl] < CAP
        c1at2 = plsc.load_gather(c1_v, [i2i])
        keep2 = (c1at2 + r2_v[sl]) < CAP
        row = (g * 16 + lane) * E
        plsc.store_scatter(out_v, [row + i1i], g1_v[sl], mask=keep1)
        plsc.store_scatter(out_v, [row + i2i], g2_v[sl], mask=keep2)
        return _

    lax.fori_loop(0, TPW // 16, _grp, None)

    pltpu.sync_copy(out_v, out_hbm.at[pl.ds(base * E, TPW * E)])


@functools.lru_cache(maxsize=1)
def _pass2():
    return pl.kernel(
        _pass2_body,
        out_type=jax.ShapeDtypeStruct((T * E,), jnp.float32),
        mesh=plsc.VectorSubcoreMesh(core_axis_name="c", subcore_axis_name="s"),
        scratch_types=[pltpu.VMEM((TPW,), jnp.float32)] * 6
        + [pltpu.VMEM((E,), jnp.float32), pltpu.VMEM((TPW * E,), jnp.float32)],
        compiler_params=pltpu.CompilerParams(needs_layout_passes=False),
    )


def kernel(x, wg_weight):
    i1, i2, g1, g2, r1, r2, cnt, me = _pass1(x, x, wg_weight, _gumbel())
    c1tot = cnt[:E, 0]
    flat = _pass2()(i1.reshape(T), i2.reshape(T), g1.reshape(T),
                    g2.reshape(T), r1.reshape(T), r2.reshape(T), c1tot)
    combine = flat.reshape(T, E)
    l_aux = jnp.sum((me[:, 0] / T) * (c1tot / T)) * E
    return (l_aux, combine)


# speedup vs baseline: 3.1846x; 1.1083x over previous
import functools

import jax
import jax.numpy as jnp
from jax import lax
from jax.experimental import pallas as pl
from jax.experimental.pallas import tpu as pltpu
from jax.experimental.pallas import tpu_sc as plsc

T = 32768
E = 64
CAP = 512.0
BLK = 512
NBLK = T // BLK
NW = 32
TPW = T // NW


@functools.lru_cache(maxsize=1)
def _gumbel():
    u = jax.random.uniform(jax.random.key(12345), (T, E), minval=1e-6, maxval=1.0 - 1e-6)
    return jnp.transpose(-jnp.log(-jnp.log(u)))


def _pass1_body(xa_ref, xb_ref, wg_ref, gum_ref, i1_ref, i2_ref, g1_ref,
                g2_ref, r1_ref, r2_ref, cnt_ref, me_ref, s_ref, stats_ref):
    pid = pl.program_id(0)

    @pl.when(pid == 0)
    def _init():
        cnt_ref[...] = jnp.zeros((2 * E, 1), jnp.float32)
        stats_ref[...] = jnp.zeros((E, BLK), jnp.float32)
        tr = lax.broadcasted_iota(jnp.int32, (BLK, BLK), 0)
        tc = lax.broadcasted_iota(jnp.int32, (BLK, BLK), 1)
        s_ref[...] = (tr < tc).astype(jnp.bfloat16)

    la = lax.dot_general(wg_ref[...], xa_ref[...],
                         (((1,), (1,)), ((), ())),
                         preferred_element_type=jnp.float32)
    lb = lax.dot_general(wg_ref[...], xb_ref[...],
                         (((1,), (1,)), ((), ())),
                         preferred_element_type=jnp.float32)
    logits = jnp.concatenate([la, lb], axis=1)
    m = jnp.max(logits, axis=0, keepdims=True)
    ex = jnp.exp(logits - m)
    zinv = 1.0 / jnp.sum(ex, axis=0, keepdims=True)

    iota_e = lax.broadcasted_iota(jnp.int32, (E, BLK), 0)
    i1 = jnp.min(jnp.where(logits == m, iota_e, 127), axis=0, keepdims=True)
    oh1 = iota_e == i1

    noisy = jnp.where(oh1, -jnp.inf, logits + gum_ref[...])
    nm = jnp.max(noisy, axis=0, keepdims=True)
    i2 = jnp.min(jnp.where(noisy == nm, iota_e, 127), axis=0, keepdims=True)
    oh2 = iota_e == i2

    ohb = jnp.concatenate(
        [oh1.astype(jnp.bfloat16), oh2.astype(jnp.bfloat16)], axis=0)
    pre = lax.dot_general(ohb, s_ref[...], (((1,), (0,)), ((), ())),
                          preferred_element_type=jnp.float32)
    ohcat = ohb.astype(jnp.float32)
    cnt = cnt_ref[...]
    ranked = ohcat * (cnt + pre)
    r1_ref[...] = jnp.sum(ranked[:E], axis=0, keepdims=True)
    r2_ref[...] = jnp.sum(ranked[E:], axis=0, keepdims=True)

    probs = ex * zinv
    g1_ref[...] = jnp.sum(jnp.where(oh1, probs, 0.0), axis=0, keepdims=True)
    g2_ref[...] = jnp.sum(jnp.where(oh2, probs, 0.0), axis=0, keepdims=True)
    i1_ref[...] = i1.astype(jnp.float32)
    i2_ref[...] = i2.astype(jnp.float32)

    cnt_ref[...] = cnt + pre[:, BLK - 1:BLK] + ohcat[:, BLK - 1:BLK]
    stats_ref[...] = stats_ref[...] + probs

    @pl.when(pid == NBLK - 1)
    def _fin():
        me_ref[...] = jnp.sum(stats_ref[...], axis=1, keepdims=True)


_pass1 = pl.pallas_call(
    _pass1_body,
    grid=(NBLK,),
    in_specs=[
        pl.BlockSpec((BLK // 2, 1024), lambda i: (2 * i, 0)),
        pl.BlockSpec((BLK // 2, 1024), lambda i: (2 * i + 1, 0)),
        pl.BlockSpec((E, 1024), lambda i: (0, 0)),
        pl.BlockSpec((E, BLK), lambda i: (0, i)),
    ],
    out_specs=[pl.BlockSpec((1, BLK), lambda i: (0, i))] * 6
    + [pl.BlockSpec((2 * E, 1), lambda i: (0, 0)),
       pl.BlockSpec((E, 1), lambda i: (0, 0))],
    out_shape=[jax.ShapeDtypeStruct((1, T), jnp.float32)] * 6
    + [jax.ShapeDtypeStruct((2 * E, 1), jnp.float32),
       jax.ShapeDtypeStruct((E, 1), jnp.float32)],
    scratch_shapes=[pltpu.VMEM((BLK, BLK), jnp.bfloat16),
                    pltpu.VMEM((E, BLK), jnp.float32)],
)


GPC = 8
CHT = GPC * 16
NCH = TPW // CHT


def _pass2_body(i1_hbm, i2_hbm, g1_hbm, g2_hbm, r1_hbm, r2_hbm, c1tot_hbm,
                out_hbm, i1_v, i2_v, g1_v, g2_v, r1_v, r2_v, c1_v, out_v,
                in_sem, out_sem):
    wid = lax.axis_index("s") * 2 + lax.axis_index("c")
    base = wid * TPW
    cps = [
        pltpu.async_copy(i1_hbm.at[pl.ds(base, TPW)], i1_v, in_sem),
        pltpu.async_copy(i2_hbm.at[pl.ds(base, TPW)], i2_v, in_sem),
        pltpu.async_copy(g1_hbm.at[pl.ds(base, TPW)], g1_v, in_sem),
        pltpu.async_copy(g2_hbm.at[pl.ds(base, TPW)], g2_v, in_sem),
        pltpu.async_copy(r1_hbm.at[pl.ds(base, TPW)], r1_v, in_sem),
        pltpu.async_copy(r2_hbm.at[pl.ds(base, TPW)], r2_v, in_sem),
        pltpu.async_copy(c1tot_hbm, c1_v, in_sem),
    ]

    zeros16 = jnp.zeros((16,), jnp.float32)

    @plsc.parallel_loop(0, TPW * E // 16, unroll=8)
    def _zero(k):
        out_v[pl.ds(k * 16, 16)] = zeros16

    for c in cps:
        c.wait()

    lane = lax.broadcasted_iota(jnp.int32, (16,), 0)
    ocps = []
    for ch in range(NCH):

        @plsc.parallel_loop(ch * GPC, (ch + 1) * GPC, unroll=4)
        def _grp(g):
            sl = pl.ds(g * 16, 16)
            i1i = i1_v[sl].astype(jnp.int32)
            i2i = i2_v[sl].astype(jnp.int32)
            keep1 = r1_v[sl] < CAP
            c1at2 = plsc.load_gather(c1_v, [i2i])
            keep2 = (c1at2 + r2_v[sl]) < CAP
            row = (g * 16 + lane) * E
            plsc.store_scatter(out_v, [row + i1i], g1_v[sl], mask=keep1)
            plsc.store_scatter(out_v, [row + i2i], g2_v[sl], mask=keep2)

        ocps.append(pltpu.async_copy(
            out_v.at[pl.ds(ch * CHT * E, CHT * E)],
            out_hbm.at[pl.ds(base * E + ch * CHT * E, CHT * E)], out_sem))

    for c in ocps:
        c.wait()


@functools.lru_cache(maxsize=1)
def _pass2():
    return pl.kernel(
        _pass2_body,
        out_type=jax.ShapeDtypeStruct((T * E,), jnp.float32),
        mesh=plsc.VectorSubcoreMesh(core_axis_name="c", subcore_axis_name="s"),
        scratch_types=[pltpu.VMEM((TPW,), jnp.float32)] * 6
        + [pltpu.VMEM((E,), jnp.float32), pltpu.VMEM((TPW * E,), jnp.float32),
           pltpu.SemaphoreType.DMA, pltpu.SemaphoreType.DMA],
        compiler_params=pltpu.CompilerParams(needs_layout_passes=False),
    )


def kernel(x, wg_weight):
    i1, i2, g1, g2, r1, r2, cnt, me = _pass1(x, x, wg_weight, _gumbel())
    c1tot = cnt[:E, 0]
    flat = _pass2()(i1.reshape(T), i2.reshape(T), g1.reshape(T),
                    g2.reshape(T), r1.reshape(T), r2.reshape(T), c1tot)
    combine = flat.reshape(T, E)
    l_aux = jnp.sum((me[:, 0] / T) * (c1tot / T)) * E
    return (l_aux, combine)


# speedup vs baseline: 3.3800x; 1.0614x over previous
import functools

import jax
import jax.numpy as jnp
from jax import lax
from jax.experimental import pallas as pl
from jax.experimental.pallas import tpu as pltpu
from jax.experimental.pallas import tpu_sc as plsc

T = 32768
E = 64
CAP = 512.0
BLK = 512
NBLK = T // BLK
NW = 32
TPW = T // NW


@functools.lru_cache(maxsize=1)
def _gumbel():
    u = jax.random.uniform(jax.random.key(12345), (T, E), minval=1e-6, maxval=1.0 - 1e-6)
    return jnp.transpose(-jnp.log(-jnp.log(u)))


def _pass1_body(xa_ref, xb_ref, wg_ref, gum_ref, i1_ref, i2_ref, g1_ref,
                g2_ref, r1_ref, r2_ref, laux_ref, c1_ref, cnt_ref, s_ref,
                stats_ref):
    pid = pl.program_id(0)

    @pl.when(pid == 0)
    def _init():
        cnt_ref[...] = jnp.zeros((2 * E, 1), jnp.float32)
        stats_ref[...] = jnp.zeros((E, BLK), jnp.float32)
        tr = lax.broadcasted_iota(jnp.int32, (BLK, BLK), 0)
        tc = lax.broadcasted_iota(jnp.int32, (BLK, BLK), 1)
        s_ref[...] = (tr < tc).astype(jnp.bfloat16)

    la = lax.dot_general(wg_ref[...], xa_ref[...],
                         (((1,), (1,)), ((), ())),
                         preferred_element_type=jnp.float32)
    lb = lax.dot_general(wg_ref[...], xb_ref[...],
                         (((1,), (1,)), ((), ())),
                         preferred_element_type=jnp.float32)
    logits = jnp.concatenate([la, lb], axis=1)
    m = jnp.max(logits, axis=0, keepdims=True)
    ex = jnp.exp(logits - m)
    zinv = 1.0 / jnp.sum(ex, axis=0, keepdims=True)

    iota_e = lax.broadcasted_iota(jnp.int32, (E, BLK), 0)
    i1 = jnp.min(jnp.where(logits == m, iota_e, 127), axis=0, keepdims=True)
    oh1 = iota_e == i1

    noisy = jnp.where(oh1, -jnp.inf, logits + gum_ref[...])
    nm = jnp.max(noisy, axis=0, keepdims=True)
    i2 = jnp.min(jnp.where(noisy == nm, iota_e, 127), axis=0, keepdims=True)
    oh2 = iota_e == i2

    ohb = jnp.concatenate(
        [oh1.astype(jnp.bfloat16), oh2.astype(jnp.bfloat16)], axis=0)
    pre = lax.dot_general(ohb, s_ref[...], (((1,), (0,)), ((), ())),
                          preferred_element_type=jnp.float32)
    ohcat = ohb.astype(jnp.float32)
    cnt = cnt_ref[...]
    ranked = ohcat * (cnt + pre)
    r1_ref[...] = jnp.sum(ranked[:E], axis=0, keepdims=True)
    r2_ref[...] = jnp.sum(ranked[E:], axis=0, keepdims=True)

    probs = ex * zinv
    g1_ref[...] = jnp.sum(jnp.where(oh1, probs, 0.0), axis=0, keepdims=True)
    g2_ref[...] = jnp.sum(jnp.where(oh2, probs, 0.0), axis=0, keepdims=True)
    i1_ref[...] = i1.astype(jnp.float32)
    i2_ref[...] = i2.astype(jnp.float32)

    newcnt = cnt + pre[:, BLK - 1:BLK] + ohcat[:, BLK - 1:BLK]
    cnt_ref[...] = newcnt
    stats_ref[...] = stats_ref[...] + probs

    @pl.when(pid == NBLK - 1)
    def _fin():
        me = jnp.sum(stats_ref[...], axis=1, keepdims=True)
        c1 = newcnt[:E]
        laux_ref[...] = jnp.sum(me * c1, axis=0, keepdims=True) * (
            float(E) / (float(T) * float(T)))
        c1_ref[...] = c1


_pass1 = pl.pallas_call(
    _pass1_body,
    grid=(NBLK,),
    in_specs=[
        pl.BlockSpec((BLK // 2, 1024), lambda i: (2 * i, 0)),
        pl.BlockSpec((BLK // 2, 1024), lambda i: (2 * i + 1, 0)),
        pl.BlockSpec((E, 1024), lambda i: (0, 0)),
        pl.BlockSpec((E, BLK), lambda i: (0, i)),
    ],
    out_specs=[pl.BlockSpec((1, BLK), lambda i: (0, i))] * 6
    + [pl.BlockSpec((1, 1), lambda i: (0, 0)),
       pl.BlockSpec((E, 1), lambda i: (0, 0))],
    out_shape=[jax.ShapeDtypeStruct((1, T), jnp.float32)] * 6
    + [jax.ShapeDtypeStruct((1, 1), jnp.float32),
       jax.ShapeDtypeStruct((E, 1), jnp.float32)],
    scratch_shapes=[pltpu.VMEM((2 * E, 1), jnp.float32),
                    pltpu.VMEM((BLK, BLK), jnp.bfloat16),
                    pltpu.VMEM((E, BLK), jnp.float32)],
)


GPC = 8
CHT = GPC * 16
NCH = TPW // CHT


def _pass2_body(i1_hbm, i2_hbm, g1_hbm, g2_hbm, r1_hbm, r2_hbm, c1tot_hbm,
                out_hbm, i1_v, i2_v, g1_v, g2_v, r1_v, r2_v, c1_v, ov0, ov1,
                in_sem, out_sem):
    wid = lax.axis_index("s") * 2 + lax.axis_index("c")
    base = wid * TPW
    cps = [
        pltpu.async_copy(i1_hbm.at[0, pl.ds(base, TPW)], i1_v, in_sem),
        pltpu.async_copy(i2_hbm.at[0, pl.ds(base, TPW)], i2_v, in_sem),
        pltpu.async_copy(g1_hbm.at[0, pl.ds(base, TPW)], g1_v, in_sem),
        pltpu.async_copy(g2_hbm.at[0, pl.ds(base, TPW)], g2_v, in_sem),
        pltpu.async_copy(r1_hbm.at[0, pl.ds(base, TPW)], r1_v, in_sem),
        pltpu.async_copy(r2_hbm.at[0, pl.ds(base, TPW)], r2_v, in_sem),
        pltpu.async_copy(c1tot_hbm, c1_v, in_sem),
    ]

    zeros16 = jnp.zeros((16,), jnp.float32)

    def _zero_buf(buf):
        @plsc.parallel_loop(0, CHT, unroll=8)
        def _z(k):
            for j in range(E // 16):
                buf[k, pl.ds(j * 16, 16)] = zeros16

    _zero_buf(ov0)
    _zero_buf(ov1)

    for c in cps:
        c.wait()

    lane = lax.broadcasted_iota(jnp.int32, (16,), 0)
    zidx = jnp.zeros((16,), jnp.int32)
    bufs = [ov0, ov1]
    ocps = []
    for ch in range(NCH):
        buf = bufs[ch % 2]
        if ch >= 2:
            ocps[ch - 2].wait()
            _zero_buf(buf)

        @plsc.parallel_loop(0, GPC, unroll=4)
        def _grp(gl):
            sl = pl.ds((ch * GPC + gl) * 16, 16)
            i1i = i1_v[sl].astype(jnp.int32)
            i2i = i2_v[sl].astype(jnp.int32)
            keep1 = r1_v[sl] < CAP
            c1at2 = plsc.load_gather(c1_v, [i2i, zidx])
            keep2 = (c1at2 + r2_v[sl]) < CAP
            tok = gl * 16 + lane
            plsc.store_scatter(buf, [tok, i1i], g1_v[sl], mask=keep1)
            plsc.store_scatter(buf, [tok, i2i], g2_v[sl], mask=keep2)

        ocps.append(pltpu.async_copy(
            buf, out_hbm.at[pl.ds(base + ch * CHT, CHT)], out_sem))

    ocps[NCH - 2].wait()
    ocps[NCH - 1].wait()


@functools.lru_cache(maxsize=1)
def _pass2():
    return pl.kernel(
        _pass2_body,
        out_type=jax.ShapeDtypeStruct((T, E), jnp.float32),
        mesh=plsc.VectorSubcoreMesh(core_axis_name="c", subcore_axis_name="s"),
        scratch_types=[pltpu.VMEM((TPW,), jnp.float32)] * 6
        + [pltpu.VMEM((E, 1), jnp.float32),
           pltpu.VMEM((CHT, E), jnp.float32),
           pltpu.VMEM((CHT, E), jnp.float32),
           pltpu.SemaphoreType.DMA, pltpu.SemaphoreType.DMA],
        compiler_params=pltpu.CompilerParams(needs_layout_passes=False),
    )


def kernel(x, wg_weight):
    i1, i2, g1, g2, r1, r2, laux, c1 = _pass1(x, x, wg_weight, _gumbel())
    combine = _pass2()(i1, i2, g1, g2, r1, r2, c1)
    return (laux[0, 0], combine)


# speedup vs baseline: 4.3320x; 1.2816x over previous
import functools

import jax
import jax.numpy as jnp
from jax import lax
from jax.experimental import pallas as pl
from jax.experimental.pallas import tpu as pltpu
from jax.experimental.pallas import tpu_sc as plsc

T = 32768
E = 64
CAP = 512.0
BLK = 512
NBLK = T // BLK
NW = 32
TPW = T // NW


@functools.lru_cache(maxsize=1)
def _gumbel():
    with jax.ensure_compile_time_eval():
        u = jax.random.uniform(jax.random.key(12345), (T, E),
                               minval=1e-6, maxval=1.0 - 1e-6)
        return jnp.transpose(-jnp.log(-jnp.log(u)))


def _pass1_body(xa_ref, xb_ref, wg_ref, gum_ref, i1_ref, i2_ref, g1_ref,
                g2_ref, r1_ref, r2_ref, laux_ref, c1_ref, cnt_ref, s_ref,
                stats_ref):
    pid = pl.program_id(0)

    @pl.when(pid == 0)
    def _init():
        cnt_ref[...] = jnp.zeros((2 * E, 1), jnp.float32)
        stats_ref[...] = jnp.zeros((E, BLK), jnp.float32)
        tr = lax.broadcasted_iota(jnp.int32, (BLK, BLK), 0)
        tc = lax.broadcasted_iota(jnp.int32, (BLK, BLK), 1)
        s_ref[...] = (tr < tc).astype(jnp.bfloat16)

    la = lax.dot_general(wg_ref[...], xa_ref[...],
                         (((1,), (1,)), ((), ())),
                         preferred_element_type=jnp.float32)
    lb = lax.dot_general(wg_ref[...], xb_ref[...],
                         (((1,), (1,)), ((), ())),
                         preferred_element_type=jnp.float32)
    logits = jnp.concatenate([la, lb], axis=1)
    m = jnp.max(logits, axis=0, keepdims=True)
    ex = jnp.exp(logits - m)
    zinv = 1.0 / jnp.sum(ex, axis=0, keepdims=True)

    iota_e = lax.broadcasted_iota(jnp.int32, (E, BLK), 0)
    i1 = jnp.min(jnp.where(logits == m, iota_e, 127), axis=0, keepdims=True)
    oh1 = iota_e == i1

    noisy = jnp.where(oh1, -jnp.inf, logits + gum_ref[...])
    nm = jnp.max(noisy, axis=0, keepdims=True)
    i2 = jnp.min(jnp.where(noisy == nm, iota_e, 127), axis=0, keepdims=True)
    oh2 = iota_e == i2

    ohb = jnp.concatenate(
        [oh1.astype(jnp.bfloat16), oh2.astype(jnp.bfloat16)], axis=0)
    pre = lax.dot_general(ohb, s_ref[...], (((1,), (0,)), ((), ())),
                          preferred_element_type=jnp.float32)
    ohcat = ohb.astype(jnp.float32)
    cnt = cnt_ref[...]
    ranked = ohcat * (cnt + pre)
    r1_ref[...] = jnp.sum(ranked[:E], axis=0, keepdims=True)
    r2_ref[...] = jnp.sum(ranked[E:], axis=0, keepdims=True)

    probs = ex * zinv
    g1_ref[...] = jnp.sum(jnp.where(oh1, probs, 0.0), axis=0, keepdims=True)
    g2_ref[...] = jnp.sum(jnp.where(oh2, probs, 0.0), axis=0, keepdims=True)
    i1_ref[...] = i1.astype(jnp.float32)
    i2_ref[...] = i2.astype(jnp.float32)

    newcnt = cnt + pre[:, BLK - 1:BLK] + ohcat[:, BLK - 1:BLK]
    cnt_ref[...] = newcnt
    stats_ref[...] = stats_ref[...] + probs

    @pl.when(pid == NBLK - 1)
    def _fin():
        me = jnp.sum(stats_ref[...], axis=1, keepdims=True)
        c1 = newcnt[:E]
        laux_ref[...] = jnp.sum(me * c1, axis=0, keepdims=True) * (
            float(E) / (float(T) * float(T)))
        c1_ref[...] = c1


_pass1 = pl.pallas_call(
    _pass1_body,
    grid=(NBLK,),
    in_specs=[
        pl.BlockSpec((BLK // 2, 1024), lambda i: (2 * i, 0)),
        pl.BlockSpec((BLK // 2, 1024), lambda i: (2 * i + 1, 0)),
        pl.BlockSpec((E, 1024), lambda i: (0, 0)),
        pl.BlockSpec((E, BLK), lambda i: (0, i)),
    ],
    out_specs=[pl.BlockSpec((1, BLK), lambda i: (0, i))] * 6
    + [pl.BlockSpec((1, 1), lambda i: (0, 0)),
       pl.BlockSpec((E, 1), lambda i: (0, 0))],
    out_shape=[jax.ShapeDtypeStruct((1, T), jnp.float32)] * 6
    + [jax.ShapeDtypeStruct((1, 1), jnp.float32),
       jax.ShapeDtypeStruct((E, 1), jnp.float32)],
    scratch_shapes=[pltpu.VMEM((2 * E, 1), jnp.float32),
                    pltpu.VMEM((BLK, BLK), jnp.bfloat16),
                    pltpu.VMEM((E, BLK), jnp.float32)],
)


GPC = 8
CHT = GPC * 16
NCH = TPW // CHT


def _pass2_body(i1_hbm, i2_hbm, g1_hbm, g2_hbm, r1_hbm, r2_hbm, c1tot_hbm,
                out_hbm, i1_v, i2_v, g1_v, g2_v, r1_v, r2_v, c1_v, ov0, ov1,
                in_sem, out_sem):
    wid = lax.axis_index("s") * 2 + lax.axis_index("c")
    base = wid * TPW
    cps = [
        pltpu.async_copy(i1_hbm.at[0, pl.ds(base, TPW)], i1_v, in_sem),
        pltpu.async_copy(i2_hbm.at[0, pl.ds(base, TPW)], i2_v, in_sem),
        pltpu.async_copy(g1_hbm.at[0, pl.ds(base, TPW)], g1_v, in_sem),
        pltpu.async_copy(g2_hbm.at[0, pl.ds(base, TPW)], g2_v, in_sem),
        pltpu.async_copy(r1_hbm.at[0, pl.ds(base, TPW)], r1_v, in_sem),
        pltpu.async_copy(r2_hbm.at[0, pl.ds(base, TPW)], r2_v, in_sem),
        pltpu.async_copy(c1tot_hbm, c1_v, in_sem),
    ]

    zeros16 = jnp.zeros((16,), jnp.float32)

    def _zero_buf(buf):
        @plsc.parallel_loop(0, CHT, unroll=8)
        def _z(k):
            for j in range(E // 16):
                buf[k, pl.ds(j * 16, 16)] = zeros16

    _zero_buf(ov0)
    _zero_buf(ov1)

    for c in cps:
        c.wait()

    lane = lax.broadcasted_iota(jnp.int32, (16,), 0)
    zidx = jnp.zeros((16,), jnp.int32)
    bufs = [ov0, ov1]
    ocps = []
    for ch in range(NCH):
        buf = bufs[ch % 2]
        if ch >= 2:
            ocps[ch - 2].wait()
            _zero_buf(buf)

        @plsc.parallel_loop(0, GPC, unroll=4)
        def _grp(gl):
            sl = pl.ds((ch * GPC + gl) * 16, 16)
            i1i = i1_v[sl].astype(jnp.int32)
            i2i = i2_v[sl].astype(jnp.int32)
            keep1 = r1_v[sl] < CAP
            c1at2 = plsc.load_gather(c1_v, [i2i, zidx])
            keep2 = (c1at2 + r2_v[sl]) < CAP
            tok = gl * 16 + lane
            plsc.store_scatter(buf, [tok, i1i], g1_v[sl], mask=keep1)
            plsc.store_scatter(buf, [tok, i2i], g2_v[sl], mask=keep2)

        ocps.append(pltpu.async_copy(
            buf, out_hbm.at[pl.ds(base + ch * CHT, CHT)], out_sem))

    ocps[NCH - 2].wait()
    ocps[NCH - 1].wait()


@functools.lru_cache(maxsize=1)
def _pass2():
    return pl.kernel(
        _pass2_body,
        out_type=jax.ShapeDtypeStruct((T, E), jnp.float32),
        mesh=plsc.VectorSubcoreMesh(core_axis_name="c", subcore_axis_name="s"),
        scratch_types=[pltpu.VMEM((TPW,), jnp.float32)] * 6
        + [pltpu.VMEM((E, 1), jnp.float32),
           pltpu.VMEM((CHT, E), jnp.float32),
           pltpu.VMEM((CHT, E), jnp.float32),
           pltpu.SemaphoreType.DMA, pltpu.SemaphoreType.DMA],
        compiler_params=pltpu.CompilerParams(needs_layout_passes=False),
    )


def kernel(x, wg_weight):
    i1, i2, g1, g2, r1, r2, laux, c1 = _pass1(x, x, wg_weight, _gumbel())
    combine = _pass2()(i1, i2, g1, g2, r1, r2, c1)
    return (laux[0, 0], combine)


# speedup vs baseline: 5.1947x; 1.1991x over previous
import functools

import jax
import jax.numpy as jnp
from jax import lax
from jax.experimental import pallas as pl
from jax.experimental.pallas import tpu as pltpu
from jax.experimental.pallas import tpu_sc as plsc

T = 32768
E = 64
CAP = 512.0
BLK = 1024
NBLK = T // BLK
NW = 32
TPW = T // NW


@functools.lru_cache(maxsize=1)
def _gumbel():
    with jax.ensure_compile_time_eval():
        u = jax.random.uniform(jax.random.key(12345), (T, E),
                               minval=1e-6, maxval=1.0 - 1e-6)
        return jnp.transpose(-jnp.log(-jnp.log(u)))


def _pass1_body(xa_ref, xb_ref, wg_ref, gum_ref, i1_ref, i2_ref, g1_ref,
                g2_ref, r1_ref, r2_ref, laux_ref, c1_ref, cnt_ref, s_ref,
                stats_ref):
    pid = pl.program_id(0)

    @pl.when(pid == 0)
    def _init():
        cnt_ref[...] = jnp.zeros((2 * E, 1), jnp.float32)
        stats_ref[...] = jnp.zeros((E, BLK), jnp.float32)
        tr = lax.broadcasted_iota(jnp.int32, (BLK, BLK), 0)
        tc = lax.broadcasted_iota(jnp.int32, (BLK, BLK), 1)
        s_ref[...] = (tr < tc).astype(jnp.bfloat16)

    la = lax.dot_general(wg_ref[...], xa_ref[...],
                         (((1,), (1,)), ((), ())),
                         preferred_element_type=jnp.float32)
    lb = lax.dot_general(wg_ref[...], xb_ref[...],
                         (((1,), (1,)), ((), ())),
                         preferred_element_type=jnp.float32)
    logits = jnp.concatenate([la, lb], axis=1)
    m = jnp.max(logits, axis=0, keepdims=True)
    ex = jnp.exp(logits - m)
    zinv = 1.0 / jnp.sum(ex, axis=0, keepdims=True)

    iota_e = lax.broadcasted_iota(jnp.int32, (E, BLK), 0)
    i1 = jnp.min(jnp.where(logits == m, iota_e, 127), axis=0, keepdims=True)
    oh1 = iota_e == i1

    noisy = jnp.where(oh1, -jnp.inf, logits + gum_ref[...])
    nm = jnp.max(noisy, axis=0, keepdims=True)
    i2 = jnp.min(jnp.where(noisy == nm, iota_e, 127), axis=0, keepdims=True)
    oh2 = iota_e == i2

    ohb = jnp.concatenate(
        [oh1.astype(jnp.bfloat16), oh2.astype(jnp.bfloat16)], axis=0)
    pre = lax.dot_general(ohb, s_ref[...], (((1,), (0,)), ((), ())),
                          preferred_element_type=jnp.float32)
    ohcat = ohb.astype(jnp.float32)
    cnt = cnt_ref[...]
    ranked = ohcat * (cnt + pre)
    r1_ref[...] = jnp.sum(ranked[:E], axis=0, keepdims=True)
    r2_ref[...] = jnp.sum(ranked[E:], axis=0, keepdims=True)

    probs = ex * zinv
    g1_ref[...] = jnp.sum(jnp.where(oh1, probs, 0.0), axis=0, keepdims=True)
    g2_ref[...] = jnp.sum(jnp.where(oh2, probs, 0.0), axis=0, keepdims=True)
    i1_ref[...] = i1.astype(jnp.float32)
    i2_ref[...] = i2.astype(jnp.float32)

    newcnt = cnt + pre[:, BLK - 1:BLK] + ohcat[:, BLK - 1:BLK]
    cnt_ref[...] = newcnt
    stats_ref[...] = stats_ref[...] + probs

    @pl.when(pid == NBLK - 1)
    def _fin():
        me = jnp.sum(stats_ref[...], axis=1, keepdims=True)
        c1 = newcnt[:E]
        laux_ref[...] = jnp.sum(me * c1, axis=0, keepdims=True) * (
            float(E) / (float(T) * float(T)))
        c1_ref[...] = c1


_pass1 = pl.pallas_call(
    _pass1_body,
    grid=(NBLK,),
    in_specs=[
        pl.BlockSpec((BLK // 2, 1024), lambda i: (2 * i, 0)),
        pl.BlockSpec((BLK // 2, 1024), lambda i: (2 * i + 1, 0)),
        pl.BlockSpec((E, 1024), lambda i: (0, 0)),
        pl.BlockSpec((E, BLK), lambda i: (0, i)),
    ],
    out_specs=[pl.BlockSpec((1, BLK), lambda i: (0, i))] * 6
    + [pl.BlockSpec((1, 1), lambda i: (0, 0)),
       pl.BlockSpec((E, 1), lambda i: (0, 0))],
    out_shape=[jax.ShapeDtypeStruct((1, T), jnp.float32)] * 6
    + [jax.ShapeDtypeStruct((1, 1), jnp.float32),
       jax.ShapeDtypeStruct((E, 1), jnp.float32)],
    scratch_shapes=[pltpu.VMEM((2 * E, 1), jnp.float32),
                    pltpu.VMEM((BLK, BLK), jnp.bfloat16),
                    pltpu.VMEM((E, BLK), jnp.float32)],
)


GPC = 8
CHT = GPC * 16
NCH = TPW // CHT


def _pass2_body(i1_hbm, i2_hbm, g1_hbm, g2_hbm, r1_hbm, r2_hbm, c1tot_hbm,
                out_hbm, i1_v, i2_v, g1_v, g2_v, r1_v, r2_v, c1_v, ov0, ov1,
                in_sem, out_sem):
    wid = lax.axis_index("s") * 2 + lax.axis_index("c")
    base = wid * TPW
    cps = [
        pltpu.async_copy(i1_hbm.at[0, pl.ds(base, TPW)], i1_v, in_sem),
        pltpu.async_copy(i2_hbm.at[0, pl.ds(base, TPW)], i2_v, in_sem),
        pltpu.async_copy(g1_hbm.at[0, pl.ds(base, TPW)], g1_v, in_sem),
        pltpu.async_copy(g2_hbm.at[0, pl.ds(base, TPW)], g2_v, in_sem),
        pltpu.async_copy(r1_hbm.at[0, pl.ds(base, TPW)], r1_v, in_sem),
        pltpu.async_copy(r2_hbm.at[0, pl.ds(base, TPW)], r2_v, in_sem),
        pltpu.async_copy(c1tot_hbm, c1_v, in_sem),
    ]

    zeros16 = jnp.zeros((16,), jnp.float32)

    def _zero_buf(buf):
        @plsc.parallel_loop(0, CHT, unroll=8)
        def _z(k):
            for j in range(E // 16):
                buf[k, pl.ds(j * 16, 16)] = zeros16

    _zero_buf(ov0)
    _zero_buf(ov1)

    for c in cps:
        c.wait()

    lane = lax.broadcasted_iota(jnp.int32, (16,), 0)
    zidx = jnp.zeros((16,), jnp.int32)
    bufs = [ov0, ov1]
    ocps = []
    for ch in range(NCH):
        buf = bufs[ch % 2]
        if ch >= 2:
            ocps[ch - 2].wait()
            _zero_buf(buf)

        @plsc.parallel_loop(0, GPC, unroll=4)
        def _grp(gl):
            sl = pl.ds((ch * GPC + gl) * 16, 16)
            i1i = i1_v[sl].astype(jnp.int32)
            i2i = i2_v[sl].astype(jnp.int32)
            keep1 = r1_v[sl] < CAP
            c1at2 = plsc.load_gather(c1_v, [i2i, zidx])
            keep2 = (c1at2 + r2_v[sl]) < CAP
            tok = gl * 16 + lane
            plsc.store_scatter(buf, [tok, i1i], g1_v[sl], mask=keep1)
            plsc.store_scatter(buf, [tok, i2i], g2_v[sl], mask=keep2)

        ocps.append(pltpu.async_copy(
            buf, out_hbm.at[pl.ds(base + ch * CHT, CHT)], out_sem))

    ocps[NCH - 2].wait()
    ocps[NCH - 1].wait()


@functools.lru_cache(maxsize=1)
def _pass2():
    return pl.kernel(
        _pass2_body,
        out_type=jax.ShapeDtypeStruct((T, E), jnp.float32),
        mesh=plsc.VectorSubcoreMesh(core_axis_name="c", subcore_axis_name="s"),
        scratch_types=[pltpu.VMEM((TPW,), jnp.float32)] * 6
        + [pltpu.VMEM((E, 1), jnp.float32),
           pltpu.VMEM((CHT, E), jnp.float32),
           pltpu.VMEM((CHT, E), jnp.float32),
           pltpu.SemaphoreType.DMA, pltpu.SemaphoreType.DMA],
        compiler_params=pltpu.CompilerParams(needs_layout_passes=False),
    )


def kernel(x, wg_weight):
    i1, i2, g1, g2, r1, r2, laux, c1 = _pass1(x, x, wg_weight, _gumbel())
    combine = _pass2()(i1, i2, g1, g2, r1, r2, c1)
    return (laux[0, 0], combine)


# speedup vs baseline: 5.3527x; 1.0304x over previous
import functools

import jax
import jax.numpy as jnp
from jax import lax
from jax.experimental import pallas as pl
from jax.experimental.pallas import tpu as pltpu
from jax.experimental.pallas import tpu_sc as plsc

T = 32768
E = 64
CAP = 512.0
BLK = 2048
NBLK = T // BLK
NW = 32
TPW = T // NW


@functools.lru_cache(maxsize=1)
def _gumbel():
    with jax.ensure_compile_time_eval():
        u = jax.random.uniform(jax.random.key(12345), (T, E),
                               minval=1e-6, maxval=1.0 - 1e-6)
        return jnp.transpose(-jnp.log(-jnp.log(u)))


def _pass1_body(xa_ref, xb_ref, wg_ref, gum_ref, i1_ref, i2_ref, g1_ref,
                g2_ref, r1_ref, r2_ref, laux_ref, c1_ref, cnt_ref, s_ref,
                stats_ref):
    pid = pl.program_id(0)

    @pl.when(pid == 0)
    def _init():
        cnt_ref[...] = jnp.zeros((2 * E, 1), jnp.float32)
        stats_ref[...] = jnp.zeros((E, BLK), jnp.float32)
        tr = lax.broadcasted_iota(jnp.int32, (BLK, BLK), 0)
        tc = lax.broadcasted_iota(jnp.int32, (BLK, BLK), 1)
        s_ref[...] = (tr < tc).astype(jnp.bfloat16)

    la = lax.dot_general(wg_ref[...], xa_ref[...],
                         (((1,), (1,)), ((), ())),
                         preferred_element_type=jnp.float32)
    lb = lax.dot_general(wg_ref[...], xb_ref[...],
                         (((1,), (1,)), ((), ())),
                         preferred_element_type=jnp.float32)
    logits = jnp.concatenate([la, lb], axis=1)
    m = jnp.max(logits, axis=0, keepdims=True)
    ex = jnp.exp(logits - m)
    zinv = 1.0 / jnp.sum(ex, axis=0, keepdims=True)

    iota_e = lax.broadcasted_iota(jnp.int32, (E, BLK), 0)
    i1 = jnp.min(jnp.where(logits == m, iota_e, 127), axis=0, keepdims=True)
    oh1 = iota_e == i1

    noisy = jnp.where(oh1, -jnp.inf, logits + gum_ref[...])
    nm = jnp.max(noisy, axis=0, keepdims=True)
    i2 = jnp.min(jnp.where(noisy == nm, iota_e, 127), axis=0, keepdims=True)
    oh2 = iota_e == i2

    ohb = jnp.concatenate(
        [oh1.astype(jnp.bfloat16), oh2.astype(jnp.bfloat16)], axis=0)
    pre = lax.dot_general(ohb, s_ref[...], (((1,), (0,)), ((), ())),
                          preferred_element_type=jnp.float32)
    ohcat = ohb.astype(jnp.float32)
    cnt = cnt_ref[...]
    ranked = ohcat * (cnt + pre)
    r1_ref[...] = jnp.sum(ranked[:E], axis=0, keepdims=True)
    r2_ref[...] = jnp.sum(ranked[E:], axis=0, keepdims=True)

    probs = ex * zinv
    g1_ref[...] = jnp.sum(jnp.where(oh1, probs, 0.0), axis=0, keepdims=True)
    g2_ref[...] = jnp.sum(jnp.where(oh2, probs, 0.0), axis=0, keepdims=True)
    i1_ref[...] = i1.astype(jnp.float32)
    i2_ref[...] = i2.astype(jnp.float32)

    newcnt = cnt + pre[:, BLK - 1:BLK] + ohcat[:, BLK - 1:BLK]
    cnt_ref[...] = newcnt
    stats_ref[...] = stats_ref[...] + probs

    @pl.when(pid == NBLK - 1)
    def _fin():
        me = jnp.sum(stats_ref[...], axis=1, keepdims=True)
        c1 = newcnt[:E]
        laux_ref[...] = jnp.sum(me * c1, axis=0, keepdims=True) * (
            float(E) / (float(T) * float(T)))
        c1_ref[...] = c1


_pass1 = pl.pallas_call(
    _pass1_body,
    grid=(NBLK,),
    in_specs=[
        pl.BlockSpec((BLK // 2, 1024), lambda i: (2 * i, 0)),
        pl.BlockSpec((BLK // 2, 1024), lambda i: (2 * i + 1, 0)),
        pl.BlockSpec((E, 1024), lambda i: (0, 0)),
        pl.BlockSpec((E, BLK), lambda i: (0, i)),
    ],
    out_specs=[pl.BlockSpec((1, BLK), lambda i: (0, i))] * 6
    + [pl.BlockSpec((1, 1), lambda i: (0, 0)),
       pl.BlockSpec((E, 1), lambda i: (0, 0))],
    out_shape=[jax.ShapeDtypeStruct((1, T), jnp.float32)] * 6
    + [jax.ShapeDtypeStruct((1, 1), jnp.float32),
       jax.ShapeDtypeStruct((E, 1), jnp.float32)],
    scratch_shapes=[pltpu.VMEM((2 * E, 1), jnp.float32),
                    pltpu.VMEM((BLK, BLK), jnp.bfloat16),
                    pltpu.VMEM((E, BLK), jnp.float32)],
)


GPC = 8
CHT = GPC * 16
NCH = TPW // CHT


def _pass2_body(i1_hbm, i2_hbm, g1_hbm, g2_hbm, r1_hbm, r2_hbm, c1tot_hbm,
                out_hbm, i1_v, i2_v, g1_v, g2_v, r1_v, r2_v, c1_v, ov0, ov1,
                in_sem, out_sem):
    wid = lax.axis_index("s") * 2 + lax.axis_index("c")
    base = wid * TPW
    cps = [
        pltpu.async_copy(i1_hbm.at[0, pl.ds(base, TPW)], i1_v, in_sem),
        pltpu.async_copy(i2_hbm.at[0, pl.ds(base, TPW)], i2_v, in_sem),
        pltpu.async_copy(g1_hbm.at[0, pl.ds(base, TPW)], g1_v, in_sem),
        pltpu.async_copy(g2_hbm.at[0, pl.ds(base, TPW)], g2_v, in_sem),
        pltpu.async_copy(r1_hbm.at[0, pl.ds(base, TPW)], r1_v, in_sem),
        pltpu.async_copy(r2_hbm.at[0, pl.ds(base, TPW)], r2_v, in_sem),
        pltpu.async_copy(c1tot_hbm, c1_v, in_sem),
    ]

    zeros16 = jnp.zeros((16,), jnp.float32)

    def _zero_buf(buf):
        @plsc.parallel_loop(0, CHT, unroll=8)
        def _z(k):
            for j in range(E // 16):
                buf[k, pl.ds(j * 16, 16)] = zeros16

    _zero_buf(ov0)
    _zero_buf(ov1)

    for c in cps:
        c.wait()

    lane = lax.broadcasted_iota(jnp.int32, (16,), 0)
    zidx = jnp.zeros((16,), jnp.int32)
    bufs = [ov0, ov1]
    ocps = []
    for ch in range(NCH):
        buf = bufs[ch % 2]
        if ch >= 2:
            ocps[ch - 2].wait()
            _zero_buf(buf)

        @plsc.parallel_loop(0, GPC, unroll=4)
        def _grp(gl):
            sl = pl.ds((ch * GPC + gl) * 16, 16)
            i1i = i1_v[sl].astype(jnp.int32)
            i2i = i2_v[sl].astype(jnp.int32)
            keep1 = r1_v[sl] < CAP
            c1at2 = plsc.load_gather(c1_v, [i2i, zidx])
            keep2 = (c1at2 + r2_v[sl]) < CAP
            tok = gl * 16 + lane
            plsc.store_scatter(buf, [tok, i1i], g1_v[sl], mask=keep1)
            plsc.store_scatter(buf, [tok, i2i], g2_v[sl], mask=keep2)

        ocps.append(pltpu.async_copy(
            buf, out_hbm.at[pl.ds(base + ch * CHT, CHT)], out_sem))

    ocps[NCH - 2].wait()
    ocps[NCH - 1].wait()


@functools.lru_cache(maxsize=1)
def _pass2():
    return pl.kernel(
        _pass2_body,
        out_type=jax.ShapeDtypeStruct((T, E), jnp.float32),
        mesh=plsc.VectorSubcoreMesh(core_axis_name="c", subcore_axis_name="s"),
        scratch_types=[pltpu.VMEM((TPW,), jnp.float32)] * 6
        + [pltpu.VMEM((E, 1), jnp.float32),
           pltpu.VMEM((CHT, E), jnp.float32),
           pltpu.VMEM((CHT, E), jnp.float32),
           pltpu.SemaphoreType.DMA, pltpu.SemaphoreType.DMA],
        compiler_params=pltpu.CompilerParams(needs_layout_passes=False),
    )


def kernel(x, wg_weight):
    i1, i2, g1, g2, r1, r2, laux, c1 = _pass1(x, x, wg_weight, _gumbel())
    combine = _pass2()(i1, i2, g1, g2, r1, r2, c1)
    return (laux[0, 0], combine)
